# baseline (speedup 1.0000x reference)
import functools

import jax
import jax.numpy as jnp
from jax import lax
from jax.experimental import pallas as pl
from jax.experimental.pallas import tpu as pltpu
from jax.experimental.pallas import tpu_sc as plsc

_B, _C, _N = 2, 192, 2048
_NB = 2
_K = 17
_EPS = 1e-5
_BN = _B * _N
_TILE = 128
_NT = _N // _TILE
_NC, _NS = 2, 16
_NW = _NC * _NS
_RPW = _BN // _NW


def _conv_body(x_ref, w_ref, b_ref, o_ref):
  o_ref[0] = (jnp.dot(w_ref[...], x_ref[0],
                      preferred_element_type=jnp.float32) + b_ref[...])


@functools.cache
def _conv(ci, co):
  return pl.pallas_call(
      _conv_body,
      grid=(_B,),
      in_specs=[
          pl.BlockSpec((1, ci, _N), lambda b: (b, 0, 0)),
          pl.BlockSpec((co, ci), lambda b: (0, 0)),
          pl.BlockSpec((co, 1), lambda b: (0, 0)),
      ],
      out_specs=pl.BlockSpec((1, co, _N), lambda b: (b, 0, 0)),
      out_shape=jax.ShapeDtypeStruct((_B, co, _N), jnp.float32),
      compiler_params=pltpu.CompilerParams(dimension_semantics=("parallel",)),
  )


def _bn(x, g, b):
  mean = jnp.mean(x, axis=(0, 2), keepdims=True)
  var = jnp.var(x, axis=(0, 2), keepdims=True)
  xh = (x - mean) / jnp.sqrt(var + _EPS)
  return xh * g[None, :, None] + b[None, :, None]


def _conv_bn(ci, co, y, w, b, g, bb):
  t0 = _conv(ci, co)(y, w, b[:, None])
  ts = jnp.einsum('oc,bcn->bon', w, y) + b[None, :, None]
  mean = jnp.mean(ts, axis=(0, 2), keepdims=True)
  var = jnp.var(ts, axis=(0, 2), keepdims=True)
  xh = (t0 - mean) / jnp.sqrt(var + _EPS)
  return xh * g[None, :, None] + bb[None, :, None]


def _topk_body(rows_ref, ht_ref, sq_ref, sqt_ref, idx_ref, dist_ref):
  b = pl.program_id(0)
  rows = rows_ref[0]
  ht = ht_ref[0]
  mm = jnp.dot(rows, ht, preferred_element_type=jnp.float32)
  dist_ref[...] = sq_ref[0] - 2.0 * mm + sqt_ref[0]
  iota = lax.broadcasted_iota(jnp.int32, (_TILE, _N), 1)
  base = b * _N
  inf = jnp.float32(jnp.inf)
  d = dist_ref[...]
  m = jnp.full((_TILE, 1), -inf, jnp.float32)
  am = jnp.full((_TILE, 1), -1, jnp.int32)
  for k in range(_K):
    sel = (d > m) | ((d == m) & (iota > am))
    m = jnp.min(jnp.where(sel, d, inf), axis=1, keepdims=True)
    am = jnp.min(jnp.where(sel & (d == m), iota, _N), axis=1, keepdims=True)
    idx_ref[0, :, k:k + 1] = am + base


_topk = pl.pallas_call(
    _topk_body,
    grid=(_B, _NT),
    in_specs=[
        pl.BlockSpec((1, _TILE, _C), lambda b, t: (b, t, 0)),
        pl.BlockSpec((1, _C, _N), lambda b, t: (b, 0, 0)),
        pl.BlockSpec((1, _TILE, 1), lambda b, t: (b, t, 0)),
        pl.BlockSpec((1, 1, _N), lambda b, t: (b, 0, 0)),
    ],
    out_specs=pl.BlockSpec((1, _TILE, _K), lambda b, t: (b, t, 0)),
    out_shape=jax.ShapeDtypeStruct((_B, _N, _K), jnp.int32),
    scratch_shapes=[pltpu.VMEM((_TILE, _N), jnp.float32)],
    compiler_params=pltpu.CompilerParams(
        dimension_semantics=("parallel", "parallel")),
)


_NBUF = 8


def _gather_max_body(h_hbm, idx_hbm, g_hbm, idxv, b0, b1, b2, b3, b4, b5, b6,
                     b7, acc, s0, s1, s2, s3, s4, s5, s6, s7):
  wid = lax.axis_index("s") * _NC + lax.axis_index("c")
  r0 = wid * _RPW
  pltpu.sync_copy(idx_hbm.at[pl.ds(r0, _RPW)], idxv)
  bufs = (b0, b1, b2, b3, b4, b5, b6, b7)
  sems = (s0, s1, s2, s3, s4, s5, s6, s7)

  for b in range(_NBUF):
    pltpu.async_copy(h_hbm.at[idxv.at[b]], bufs[b], sems[b])

  @pl.loop(0, _RPW // _NBUF)
  def _group(g):
    for b in range(_NBUF):
      p = g * _NBUF + b
      pltpu.make_async_copy(h_hbm.at[idxv.at[p]], bufs[b], sems[b]).wait()
      for j in range(_C // 16):
        sl = pl.ds(j * 16, 16)
        m = bufs[b][0, sl]
        for r in range(1, _K):
          m = jnp.maximum(m, bufs[b][r, sl])
        acc[pl.ds(p * _C + j * 16, 16)] = m

      @pl.when(p + _NBUF < _RPW)
      def _refill():
        pltpu.async_copy(h_hbm.at[idxv.at[p + _NBUF]], bufs[b], sems[b])

  pltpu.sync_copy(acc, g_hbm.at[pl.ds(r0 * _C, _RPW * _C)])


@functools.cache
def _gather_max():
  return pl.kernel(
      _gather_max_body,
      out_type=jax.ShapeDtypeStruct((_BN * _C,), jnp.float32),
      mesh=plsc.VectorSubcoreMesh(core_axis_name="c", subcore_axis_name="s"),
      scratch_types=(
          [pltpu.VMEM((_RPW, _K), jnp.int32)]
          + [pltpu.VMEM((_K, _C), jnp.float32)] * _NBUF
          + [pltpu.VMEM((_RPW * _C,), jnp.float32)]
          + [pltpu.SemaphoreType.DMA] * _NBUF
      ),
      compiler_params=pltpu.CompilerParams(use_tc_tiling_on_sc=False),
  )


@jax.jit
def kernel(x, fc1_w, fc1_b, bn1_g, bn1_b, gc_w, gc_b, bng_g, bng_b, gc_a,
           fc2_w, fc2_b, bn2_g, bn2_b, f1_w, f1_b, bf1_g, bf1_b, f_a,
           f2_w, f2_b, bf2_g, bf2_b):
  for i in range(_NB):
    shortcut = x
    h = _conv_bn(_C, _C, x, fc1_w[i], fc1_b[i], bn1_g[i], bn1_b[i])
    xt = jnp.transpose(h, (0, 2, 1))
    sq = jnp.sum(xt * xt, axis=-1, keepdims=True)
    sqt = jnp.transpose(sq, (0, 2, 1))
    idx = _topk(xt, h, sq, sqt)
    g_rows = _gather_max()(xt.reshape(_BN, _C), idx.reshape(_BN, _K))
    xj = jnp.transpose(g_rows.reshape(_B, _N, _C), (0, 2, 1)) - h
    y = jnp.stack([h, xj], axis=2).reshape(_B, 2 * _C, _N)
    y = _conv_bn(2 * _C, 2 * _C, y, gc_w[i], gc_b[i], bng_g[i], bng_b[i])
    y = jnp.where(y >= 0.0, y, gc_a[i] * y)
    h = _conv_bn(2 * _C, _C, y, fc2_w[i], fc2_b[i], bn2_g[i], bn2_b[i])
    x = h + shortcut
    shortcut = x
    h = _conv_bn(_C, 4 * _C, x, f1_w[i], f1_b[i], bf1_g[i], bf1_b[i])
    h = jnp.where(h >= 0.0, h, f_a[i] * h)
    h = _conv_bn(4 * _C, _C, h, f2_w[i], f2_b[i], bf2_g[i], bf2_b[i])
    x = h + shortcut
  return x


# speedup vs baseline: 1.2674x; 1.2674x over previous
import functools

import jax
import jax.numpy as jnp
from jax import lax
from jax.experimental import pallas as pl
from jax.experimental.pallas import tpu as pltpu
from jax.experimental.pallas import tpu_sc as plsc

_B, _C, _N = 2, 192, 2048
_NB = 2
_K = 17
_EPS = 1e-5
_BN = _B * _N
_TILE = 128
_NT = _N // _TILE
_NC, _NS = 2, 16
_NW = _NC * _NS
_RPW = _BN // _NW


def _conv_body(x_ref, w_ref, b_ref, o_ref):
  o_ref[0] = (jnp.dot(w_ref[...], x_ref[0],
                      preferred_element_type=jnp.float32) + b_ref[...])


@functools.cache
def _conv(ci, co):
  return pl.pallas_call(
      _conv_body,
      grid=(_B,),
      in_specs=[
          pl.BlockSpec((1, ci, _N), lambda b: (b, 0, 0)),
          pl.BlockSpec((co, ci), lambda b: (0, 0)),
          pl.BlockSpec((co, 1), lambda b: (0, 0)),
      ],
      out_specs=pl.BlockSpec((1, co, _N), lambda b: (b, 0, 0)),
      out_shape=jax.ShapeDtypeStruct((_B, co, _N), jnp.float32),
      compiler_params=pltpu.CompilerParams(dimension_semantics=("parallel",)),
  )


def _bn(x, g, b):
  mean = jnp.mean(x, axis=(0, 2), keepdims=True)
  var = jnp.var(x, axis=(0, 2), keepdims=True)
  xh = (x - mean) / jnp.sqrt(var + _EPS)
  return xh * g[None, :, None] + b[None, :, None]


def _conv_bn(ci, co, y, w, b, g, bb):
  t0 = _conv(ci, co)(y, w, b[:, None])
  ts = jnp.einsum('oc,bcn->bon', w, y) + b[None, :, None]
  mean = jnp.mean(ts, axis=(0, 2), keepdims=True)
  var = jnp.var(ts, axis=(0, 2), keepdims=True)
  xh = (t0 - mean) / jnp.sqrt(var + _EPS)
  return xh * g[None, :, None] + bb[None, :, None]


def _topk_body(rows_ref, ht_ref, sq_ref, sqt_ref, idx_ref, dist_ref):
  b = pl.program_id(0)
  rows = rows_ref[0]
  ht = ht_ref[0]
  mm = jnp.dot(rows, ht, preferred_element_type=jnp.float32)
  d = sq_ref[0] - 2.0 * mm + sqt_ref[0]
  iota = lax.broadcasted_iota(jnp.int32, (_TILE, _N), 1)
  dist_ref[...] = d
  base = b * _N
  for k in range(_K):
    d = dist_ref[...]
    m = jnp.min(d, axis=1, keepdims=True)
    am = jnp.min(jnp.where(d == m, iota, _N), axis=1, keepdims=True)
    idx_ref[0, :, k:k + 1] = am + base
    if k + 1 < _K:
      dist_ref[...] = jnp.where(iota == am, jnp.float32(jnp.inf), d)


_topk = pl.pallas_call(
    _topk_body,
    grid=(_B, _NT),
    in_specs=[
        pl.BlockSpec((1, _TILE, _C), lambda b, t: (b, t, 0)),
        pl.BlockSpec((1, _C, _N), lambda b, t: (b, 0, 0)),
        pl.BlockSpec((1, _TILE, 1), lambda b, t: (b, t, 0)),
        pl.BlockSpec((1, 1, _N), lambda b, t: (b, 0, 0)),
    ],
    out_specs=pl.BlockSpec((1, _TILE, _K), lambda b, t: (b, t, 0)),
    out_shape=jax.ShapeDtypeStruct((_B, _N, _K), jnp.int32),
    scratch_shapes=[pltpu.VMEM((_TILE, _N), jnp.float32)],
    compiler_params=pltpu.CompilerParams(
        dimension_semantics=("parallel", "parallel")),
)


_NBUF = 8


def _gather_max_body(h_hbm, idx_hbm, g_hbm, idxv, b0, b1, b2, b3, b4, b5, b6,
                     b7, acc, s0, s1, s2, s3, s4, s5, s6, s7):
  wid = lax.axis_index("s") * _NC + lax.axis_index("c")
  r0 = wid * _RPW
  pltpu.sync_copy(idx_hbm.at[pl.ds(r0, _RPW)], idxv)
  bufs = (b0, b1, b2, b3, b4, b5, b6, b7)
  sems = (s0, s1, s2, s3, s4, s5, s6, s7)

  for b in range(_NBUF):
    pltpu.async_copy(h_hbm.at[idxv.at[b]], bufs[b], sems[b])

  @pl.loop(0, _RPW // _NBUF)
  def _group(g):
    for b in range(_NBUF):
      p = g * _NBUF + b
      pltpu.make_async_copy(h_hbm.at[idxv.at[p]], bufs[b], sems[b]).wait()
      for j in range(_C // 16):
        sl = pl.ds(j * 16, 16)
        m = bufs[b][0, sl]
        for r in range(1, _K):
          m = jnp.maximum(m, bufs[b][r, sl])
        acc[pl.ds(p * _C + j * 16, 16)] = m

      @pl.when(p + _NBUF < _RPW)
      def _refill():
        pltpu.async_copy(h_hbm.at[idxv.at[p + _NBUF]], bufs[b], sems[b])

  pltpu.sync_copy(acc, g_hbm.at[pl.ds(r0 * _C, _RPW * _C)])


@functools.cache
def _gather_max():
  return pl.kernel(
      _gather_max_body,
      out_type=jax.ShapeDtypeStruct((_BN * _C,), jnp.float32),
      mesh=plsc.VectorSubcoreMesh(core_axis_name="c", subcore_axis_name="s"),
      scratch_types=(
          [pltpu.VMEM((_RPW, _K), jnp.int32)]
          + [pltpu.VMEM((_K, _C), jnp.float32)] * _NBUF
          + [pltpu.VMEM((_RPW * _C,), jnp.float32)]
          + [pltpu.SemaphoreType.DMA] * _NBUF
      ),
      compiler_params=pltpu.CompilerParams(use_tc_tiling_on_sc=False),
  )


@jax.jit
def kernel(x, fc1_w, fc1_b, bn1_g, bn1_b, gc_w, gc_b, bng_g, bng_b, gc_a,
           fc2_w, fc2_b, bn2_g, bn2_b, f1_w, f1_b, bf1_g, bf1_b, f_a,
           f2_w, f2_b, bf2_g, bf2_b):
  for i in range(_NB):
    shortcut = x
    h = _conv_bn(_C, _C, x, fc1_w[i], fc1_b[i], bn1_g[i], bn1_b[i])
    xt = jnp.transpose(h, (0, 2, 1))
    sq = jnp.sum(xt * xt, axis=-1, keepdims=True)
    sqt = jnp.transpose(sq, (0, 2, 1))
    idx = _topk(xt, h, sq, sqt)
    g_rows = _gather_max()(xt.reshape(_BN, _C), idx.reshape(_BN, _K))
    xj = jnp.transpose(g_rows.reshape(_B, _N, _C), (0, 2, 1)) - h
    y = jnp.stack([h, xj], axis=2).reshape(_B, 2 * _C, _N)
    y = _conv_bn(2 * _C, 2 * _C, y, gc_w[i], gc_b[i], bng_g[i], bng_b[i])
    y = jnp.where(y >= 0.0, y, gc_a[i] * y)
    h = _conv_bn(2 * _C, _C, y, fc2_w[i], fc2_b[i], bn2_g[i], bn2_b[i])
    x = h + shortcut
    shortcut = x
    h = _conv_bn(_C, 4 * _C, x, f1_w[i], f1_b[i], bf1_g[i], bf1_b[i])
    h = jnp.where(h >= 0.0, h, f_a[i] * h)
    h = _conv_bn(4 * _C, _C, h, f2_w[i], f2_b[i], bf2_g[i], bf2_b[i])
    x = h + shortcut
  return x


# speedup vs baseline: 1.4391x; 1.1355x over previous
import functools

import jax
import jax.numpy as jnp
from jax import lax
from jax.experimental import pallas as pl
from jax.experimental.pallas import tpu as pltpu
from jax.experimental.pallas import tpu_sc as plsc

_B, _C, _N = 2, 192, 2048
_NB = 2
_K = 17
_EPS = 1e-5
_BN = _B * _N
_TILE = 256
_NT = _N // _TILE
_NC, _NS = 2, 16
_NW = _NC * _NS
_RPW = _BN // _NW


def _conv_body(x_ref, w_ref, b_ref, o_ref):
  o_ref[0] = (jnp.dot(w_ref[...], x_ref[0],
                      preferred_element_type=jnp.float32) + b_ref[...])


@functools.cache
def _conv(ci, co):
  return pl.pallas_call(
      _conv_body,
      grid=(_B,),
      in_specs=[
          pl.BlockSpec((1, ci, _N), lambda b: (b, 0, 0)),
          pl.BlockSpec((co, ci), lambda b: (0, 0)),
          pl.BlockSpec((co, 1), lambda b: (0, 0)),
      ],
      out_specs=pl.BlockSpec((1, co, _N), lambda b: (b, 0, 0)),
      out_shape=jax.ShapeDtypeStruct((_B, co, _N), jnp.float32),
      compiler_params=pltpu.CompilerParams(dimension_semantics=("parallel",)),
  )


def _bn(x, g, b):
  mean = jnp.mean(x, axis=(0, 2), keepdims=True)
  var = jnp.var(x, axis=(0, 2), keepdims=True)
  xh = (x - mean) / jnp.sqrt(var + _EPS)
  return xh * g[None, :, None] + b[None, :, None]


def _conv_bn(ci, co, y, w, b, g, bb):
  t0 = _conv(ci, co)(y, w, b[:, None])
  ts = jnp.einsum('oc,bcn->bon', w, y) + b[None, :, None]
  mean = jnp.mean(ts, axis=(0, 2), keepdims=True)
  var = jnp.var(ts, axis=(0, 2), keepdims=True)
  xh = (t0 - mean) / jnp.sqrt(var + _EPS)
  return xh * g[None, :, None] + bb[None, :, None]


def _topk_body(rows_ref, ht_ref, sq_ref, sqt_ref, idx_ref, dist_ref):
  b = pl.program_id(0)
  rows = rows_ref[0]
  ht = ht_ref[0]
  mm = jnp.dot(rows, ht, preferred_element_type=jnp.float32)
  d = sq_ref[0] - 2.0 * mm + sqt_ref[0]
  iota = lax.broadcasted_iota(jnp.int32, (_TILE, _N), 1)
  dist_ref[...] = d
  base = b * _N
  for k in range(_K):
    d = dist_ref[...]
    m = jnp.min(d, axis=1, keepdims=True)
    am = jnp.min(jnp.where(d == m, iota, _N), axis=1, keepdims=True)
    idx_ref[0, :, k:k + 1] = am + base
    if k + 1 < _K:
      dist_ref[...] = jnp.where(iota == am, jnp.float32(jnp.inf), d)


_topk = pl.pallas_call(
    _topk_body,
    grid=(_B, _NT),
    in_specs=[
        pl.BlockSpec((1, _TILE, _C), lambda b, t: (b, t, 0)),
        pl.BlockSpec((1, _C, _N), lambda b, t: (b, 0, 0)),
        pl.BlockSpec((1, _TILE, 1), lambda b, t: (b, t, 0)),
        pl.BlockSpec((1, 1, _N), lambda b, t: (b, 0, 0)),
    ],
    out_specs=pl.BlockSpec((1, _TILE, _K), lambda b, t: (b, t, 0)),
    out_shape=jax.ShapeDtypeStruct((_B, _N, _K), jnp.int32),
    scratch_shapes=[pltpu.VMEM((_TILE, _N), jnp.float32)],
    compiler_params=pltpu.CompilerParams(
        dimension_semantics=("parallel", "parallel")),
)


_NBUF = 4


def _gather_max_body(h_hbm, idx_hbm, g_hbm, idxv, b0, b1, b2, b3, acc,
                     s0, s1, s2, s3):
  wid = lax.axis_index("s") * _NC + lax.axis_index("c")
  r0 = wid * _RPW
  pltpu.sync_copy(idx_hbm.at[pl.ds(r0, _RPW)], idxv)
  bufs = (b0, b1, b2, b3)
  sems = (s0, s1, s2, s3)

  for b in range(_NBUF):
    pltpu.async_copy(h_hbm.at[idxv.at[b]], bufs[b], sems[b])

  @pl.loop(0, _RPW // _NBUF)
  def _group(g):
    for b in range(_NBUF):
      p = g * _NBUF + b
      pltpu.make_async_copy(h_hbm.at[idxv.at[p]], bufs[b], sems[b]).wait()
      for j in range(_C // 16):
        sl = pl.ds(j * 16, 16)
        m = bufs[b][0, sl]
        for r in range(1, _K):
          m = jnp.maximum(m, bufs[b][r, sl])
        acc[pl.ds(p * _C + j * 16, 16)] = m

      @pl.when(p + _NBUF < _RPW)
      def _refill():
        pltpu.async_copy(h_hbm.at[idxv.at[p + _NBUF]], bufs[b], sems[b])

  pltpu.sync_copy(acc, g_hbm.at[pl.ds(r0 * _C, _RPW * _C)])


@functools.cache
def _gather_max():
  return pl.kernel(
      _gather_max_body,
      out_type=jax.ShapeDtypeStruct((_BN * _C,), jnp.float32),
      mesh=plsc.VectorSubcoreMesh(core_axis_name="c", subcore_axis_name="s"),
      scratch_types=(
          [pltpu.VMEM((_RPW, _K), jnp.int32)]
          + [pltpu.VMEM((_K, _C), jnp.float32)] * _NBUF
          + [pltpu.VMEM((_RPW * _C,), jnp.float32)]
          + [pltpu.SemaphoreType.DMA] * _NBUF
      ),
      compiler_params=pltpu.CompilerParams(use_tc_tiling_on_sc=False),
  )


@jax.jit
def kernel(x, fc1_w, fc1_b, bn1_g, bn1_b, gc_w, gc_b, bng_g, bng_b, gc_a,
           fc2_w, fc2_b, bn2_g, bn2_b, f1_w, f1_b, bf1_g, bf1_b, f_a,
           f2_w, f2_b, bf2_g, bf2_b):
  for i in range(_NB):
    shortcut = x
    h = _conv_bn(_C, _C, x, fc1_w[i], fc1_b[i], bn1_g[i], bn1_b[i])
    xt = jnp.transpose(h, (0, 2, 1))
    sq = jnp.sum(xt * xt, axis=-1, keepdims=True)
    sqt = jnp.transpose(sq, (0, 2, 1))
    idx = _topk(xt, h, sq, sqt)
    g_rows = _gather_max()(xt.reshape(_BN, _C), idx.reshape(_BN, _K))
    xj = jnp.transpose(g_rows.reshape(_B, _N, _C), (0, 2, 1)) - h
    y = jnp.stack([h, xj], axis=2).reshape(_B, 2 * _C, _N)
    y = _conv_bn(2 * _C, 2 * _C, y, gc_w[i], gc_b[i], bng_g[i], bng_b[i])
    y = jnp.where(y >= 0.0, y, gc_a[i] * y)
    h = _conv_bn(2 * _C, _C, y, fc2_w[i], fc2_b[i], bn2_g[i], bn2_b[i])
    x = h + shortcut
    shortcut = x
    h = _conv_bn(_C, 4 * _C, x, f1_w[i], f1_b[i], bf1_g[i], bf1_b[i])
    h = jnp.where(h >= 0.0, h, f_a[i] * h)
    h = _conv_bn(4 * _C, _C, h, f2_w[i], f2_b[i], bf2_g[i], bf2_b[i])
    x = h + shortcut
  return x


# speedup vs baseline: 1.4904x; 1.0357x over previous
import functools

import jax
import jax.numpy as jnp
from jax import lax
from jax.experimental import pallas as pl
from jax.experimental.pallas import tpu as pltpu
from jax.experimental.pallas import tpu_sc as plsc

_B, _C, _N = 2, 192, 2048
_NB = 2
_K = 17
_EPS = 1e-5
_BN = _B * _N
_TILE = 512
_NT = _N // _TILE
_NC, _NS = 2, 16
_NW = _NC * _NS
_RPW = _BN // _NW


def _conv_body(x_ref, w_ref, b_ref, o_ref):
  o_ref[0] = (jnp.dot(w_ref[...], x_ref[0],
                      preferred_element_type=jnp.float32) + b_ref[...])


@functools.cache
def _conv(ci, co):
  return pl.pallas_call(
      _conv_body,
      grid=(_B,),
      in_specs=[
          pl.BlockSpec((1, ci, _N), lambda b: (b, 0, 0)),
          pl.BlockSpec((co, ci), lambda b: (0, 0)),
          pl.BlockSpec((co, 1), lambda b: (0, 0)),
      ],
      out_specs=pl.BlockSpec((1, co, _N), lambda b: (b, 0, 0)),
      out_shape=jax.ShapeDtypeStruct((_B, co, _N), jnp.float32),
      compiler_params=pltpu.CompilerParams(dimension_semantics=("parallel",)),
  )


def _bn(x, g, b):
  mean = jnp.mean(x, axis=(0, 2), keepdims=True)
  var = jnp.var(x, axis=(0, 2), keepdims=True)
  xh = (x - mean) / jnp.sqrt(var + _EPS)
  return xh * g[None, :, None] + b[None, :, None]


def _conv_bn(ci, co, y, w, b, g, bb):
  t0 = _conv(ci, co)(y, w, b[:, None])
  ts = jnp.einsum('oc,bcn->bon', w, y) + b[None, :, None]
  mean = jnp.mean(ts, axis=(0, 2), keepdims=True)
  var = jnp.var(ts, axis=(0, 2), keepdims=True)
  xh = (t0 - mean) / jnp.sqrt(var + _EPS)
  return xh * g[None, :, None] + bb[None, :, None]


def _topk_body(rows_ref, ht_ref, sq_ref, sqt_ref, idx_ref, dist_ref):
  b = pl.program_id(0)
  rows = rows_ref[0]
  ht = ht_ref[0]
  mm = jnp.dot(rows, ht, preferred_element_type=jnp.float32)
  d = sq_ref[0] - 2.0 * mm + sqt_ref[0]
  iota = lax.broadcasted_iota(jnp.int32, (_TILE, _N), 1)
  dist_ref[...] = d
  base = b * _N
  for k in range(_K):
    d = dist_ref[...]
    m = jnp.min(d, axis=1, keepdims=True)
    am = jnp.min(jnp.where(d == m, iota, _N), axis=1, keepdims=True)
    idx_ref[0, :, k:k + 1] = am + base
    if k + 1 < _K:
      dist_ref[...] = jnp.where(iota == am, jnp.float32(jnp.inf), d)


_topk = pl.pallas_call(
    _topk_body,
    grid=(_B, _NT),
    in_specs=[
        pl.BlockSpec((1, _TILE, _C), lambda b, t: (b, t, 0)),
        pl.BlockSpec((1, _C, _N), lambda b, t: (b, 0, 0)),
        pl.BlockSpec((1, _TILE, 1), lambda b, t: (b, t, 0)),
        pl.BlockSpec((1, 1, _N), lambda b, t: (b, 0, 0)),
    ],
    out_specs=pl.BlockSpec((1, _TILE, _K), lambda b, t: (b, t, 0)),
    out_shape=jax.ShapeDtypeStruct((_B, _N, _K), jnp.int32),
    scratch_shapes=[pltpu.VMEM((_TILE, _N), jnp.float32)],
    compiler_params=pltpu.CompilerParams(
        dimension_semantics=("parallel", "parallel")),
)


_NBUF = 4


def _gather_max_body(h_hbm, idx_hbm, g_hbm, idxv, b0, b1, b2, b3, acc,
                     s0, s1, s2, s3):
  wid = lax.axis_index("s") * _NC + lax.axis_index("c")
  r0 = wid * _RPW
  pltpu.sync_copy(idx_hbm.at[pl.ds(r0, _RPW)], idxv)
  bufs = (b0, b1, b2, b3)
  sems = (s0, s1, s2, s3)

  for b in range(_NBUF):
    pltpu.async_copy(h_hbm.at[idxv.at[b]], bufs[b], sems[b])

  @pl.loop(0, _RPW // _NBUF)
  def _group(g):
    for b in range(_NBUF):
      p = g * _NBUF + b
      pltpu.make_async_copy(h_hbm.at[idxv.at[p]], bufs[b], sems[b]).wait()
      for j in range(_C // 16):
        sl = pl.ds(j * 16, 16)
        m = bufs[b][0, sl]
        for r in range(1, _K):
          m = jnp.maximum(m, bufs[b][r, sl])
        acc[pl.ds(p * _C + j * 16, 16)] = m

      @pl.when(p + _NBUF < _RPW)
      def _refill():
        pltpu.async_copy(h_hbm.at[idxv.at[p + _NBUF]], bufs[b], sems[b])

  pltpu.sync_copy(acc, g_hbm.at[pl.ds(r0 * _C, _RPW * _C)])


@functools.cache
def _gather_max():
  return pl.kernel(
      _gather_max_body,
      out_type=jax.ShapeDtypeStruct((_BN * _C,), jnp.float32),
      mesh=plsc.VectorSubcoreMesh(core_axis_name="c", subcore_axis_name="s"),
      scratch_types=(
          [pltpu.VMEM((_RPW, _K), jnp.int32)]
          + [pltpu.VMEM((_K, _C), jnp.float32)] * _NBUF
          + [pltpu.VMEM((_RPW * _C,), jnp.float32)]
          + [pltpu.SemaphoreType.DMA] * _NBUF
      ),
      compiler_params=pltpu.CompilerParams(use_tc_tiling_on_sc=False),
  )


@jax.jit
def kernel(x, fc1_w, fc1_b, bn1_g, bn1_b, gc_w, gc_b, bng_g, bng_b, gc_a,
           fc2_w, fc2_b, bn2_g, bn2_b, f1_w, f1_b, bf1_g, bf1_b, f_a,
           f2_w, f2_b, bf2_g, bf2_b):
  for i in range(_NB):
    shortcut = x
    h = _conv_bn(_C, _C, x, fc1_w[i], fc1_b[i], bn1_g[i], bn1_b[i])
    xt = jnp.transpose(h, (0, 2, 1))
    sq = jnp.sum(xt * xt, axis=-1, keepdims=True)
    sqt = jnp.transpose(sq, (0, 2, 1))
    idx = _topk(xt, h, sq, sqt)
    g_rows = _gather_max()(xt.reshape(_BN, _C), idx.reshape(_BN, _K))
    xj = jnp.transpose(g_rows.reshape(_B, _N, _C), (0, 2, 1)) - h
    y = jnp.stack([h, xj], axis=2).reshape(_B, 2 * _C, _N)
    y = _conv_bn(2 * _C, 2 * _C, y, gc_w[i], gc_b[i], bng_g[i], bng_b[i])
    y = jnp.where(y >= 0.0, y, gc_a[i] * y)
    h = _conv_bn(2 * _C, _C, y, fc2_w[i], fc2_b[i], bn2_g[i], bn2_b[i])
    x = h + shortcut
    shortcut = x
    h = _conv_bn(_C, 4 * _C, x, f1_w[i], f1_b[i], bf1_g[i], bf1_b[i])
    h = jnp.where(h >= 0.0, h, f_a[i] * h)
    h = _conv_bn(4 * _C, _C, h, f2_w[i], f2_b[i], bf2_g[i], bf2_b[i])
    x = h + shortcut
  return x
